# baseline (speedup 1.0000x reference)
import functools

import jax
import jax.numpy as jnp
from jax import lax
from jax.experimental import pallas as pl
from jax.experimental.pallas import tpu as pltpu
from jax.experimental.pallas import tpu_sc as plsc

N = 10000
E = 320000
D = 128
G = 32
H = 128
A = 2000

NC = 2
NS = 16
NW = NC * NS
CH = 80
EPW = 10240
EPAD = EPW * NW
NCHUNK = EPW // CH
NIT = NCHUNK // 2
NP = 10240
RPT = NP // NS
ZR = 128
NZ = RPT // ZR
APAD = 2048
APW = APAD // NW

_HI = jax.lax.Precision.HIGHEST
f32 = jnp.float32

@functools.lru_cache(maxsize=1)
def _sc_mesh():
    return plsc.VectorSubcoreMesh(core_axis_name="c", subcore_axis_name="s",
                                  num_cores=NC, num_subcores=NS)


def _mm(a, b):
    return jnp.matmul(a, b, precision=_HI)


@functools.lru_cache(maxsize=1)
def _sc_segment_sum_kernel():
    return pl.kernel(
        _sc_segment_sum_body,
        out_type=jax.ShapeDtypeStruct((NC, NP, D), f32),
        mesh=_sc_mesh(),
        compiler_params=pltpu.CompilerParams(needs_layout_passes=False),
        scratch_types=[
            pltpu.VMEM((2, 2, CH), jnp.int32),
            pltpu.VMEM((CH, D), f32),
            pltpu.VMEM((CH, D), f32),
            pltpu.VMEM((ZR, D), f32),
            pltpu.VMEM_SHARED((NP, D), f32),
            pltpu.SemaphoreType.DMA,
            pltpu.SemaphoreType.DMA,
        ],
    )


def _sc_segment_sum_body(y_hbm, src_hbm, dst_hbm, out_hbm,
                         idx, rows0, rows1, buf, acc, sem0, sem1):
    cid = lax.axis_index("c")
    sid = lax.axis_index("s")
    rows = (rows0, rows1)
    sems = (sem0, sem1)
    ebase = (cid * NS + sid) * EPW

    def issue(c, b):
        base = ebase + c * CH
        pltpu.sync_copy(src_hbm.at[pl.ds(base, CH)], idx.at[b, 0])
        pltpu.sync_copy(dst_hbm.at[pl.ds(base, CH)], idx.at[b, 1])
        pltpu.async_copy(y_hbm.at[idx.at[b, 0]], rows[b], sems[b])

    def wait(b):
        pltpu.make_async_copy(y_hbm.at[idx.at[b, 0]], rows[b], sems[b]).wait()

    def scatter(b):
        pltpu.sync_copy(rows[b], acc.at[idx.at[b, 1]], add=True)

    issue(0, 0)
    zv = jnp.zeros((16,), f32)

    def zrow(r, _):
        for jj in range(8):
            buf[r, pl.ds(16 * jj, 16)] = zv
        return 0

    lax.fori_loop(0, ZR, zrow, 0)
    for z in range(NZ):
        pltpu.sync_copy(buf, acc.at[pl.ds(sid * RPT + z * ZR, ZR)])
    plsc.subcore_barrier()

    def step(i, _):
        issue(2 * i + 1, 1)
        wait(0)
        scatter(0)

        @pl.when(i < NIT - 1)
        def _():
            issue(2 * i + 2, 0)

        wait(1)
        scatter(1)
        return 0

    lax.fori_loop(0, NIT, step, 0)
    plsc.subcore_barrier()

    for z in range(NZ):
        r0 = sid * RPT + z * ZR
        pltpu.sync_copy(acc.at[pl.ds(r0, ZR)],
                        out_hbm.at[cid, pl.ds(r0, ZR)])


@functools.lru_cache(maxsize=1)
def _sc_attack_kernel():
    return pl.kernel(
        _sc_attack_body,
        out_type=(
            jax.ShapeDtypeStruct((EPAD,), f32),
            jax.ShapeDtypeStruct((APAD, D), f32),
        ),
        mesh=_sc_mesh(),
        compiler_params=pltpu.CompilerParams(needs_layout_passes=False),
        scratch_types=[
            pltpu.VMEM((2, 2, CH), jnp.int32),
            pltpu.VMEM((CH, D), f32),
            pltpu.VMEM((CH, D), f32),
            pltpu.VMEM((CH, D), f32),
            pltpu.VMEM((CH, D), f32),
            pltpu.VMEM((CH,), f32),
            pltpu.VMEM((CH,), f32),
            pltpu.VMEM((8, 16), f32),
            pltpu.VMEM((16,), f32),
            pltpu.VMEM((APW,), jnp.int32),
            pltpu.VMEM((APW, D), f32),
            pltpu.VMEM((16, 16), f32),
            pltpu.SemaphoreType.DMA,
            pltpu.SemaphoreType.DMA,
            pltpu.SemaphoreType.DMA,
            pltpu.SemaphoreType.DMA,
            pltpu.SemaphoreType.DMA,
            pltpu.SemaphoreType.DMA,
            pltpu.SemaphoreType.DMA,
        ],
    )


def _sc_attack_body(p1_hbm, p2_hbm, asrc_hbm, adst_hbm, wa2_hbm, ba2_hbm,
                    x_hbm, aidx_hbm, q_hbm, arows_hbm,
                    idx, p1r0, p1r1, p2r0, p2r1, qout0, qout1,
                    wa2v, ba2v, aidx, arows, tbuf,
                    s1a, s1b, s2a, s2b, sqa, sqb, sal):
    cid = lax.axis_index("c")
    sid = lax.axis_index("s")
    w = cid * NS + sid
    ebase = w * EPW
    p1r = (p1r0, p1r1)
    p2r = (p2r0, p2r1)
    qout = (qout0, qout1)
    sem1 = (s1a, s1b)
    sem2 = (s2a, s2b)
    semq = (sqa, sqb)

    def issue(c, b):
        base = ebase + c * CH
        pltpu.sync_copy(asrc_hbm.at[pl.ds(base, CH)], idx.at[b, 0])
        pltpu.sync_copy(adst_hbm.at[pl.ds(base, CH)], idx.at[b, 1])
        pltpu.async_copy(p1_hbm.at[idx.at[b, 0]], p1r[b], sem1[b])
        pltpu.async_copy(p2_hbm.at[idx.at[b, 1]], p2r[b], sem2[b])

    def wait(b):
        pltpu.make_async_copy(p1_hbm.at[idx.at[b, 0]], p1r[b], sem1[b]).wait()
        pltpu.make_async_copy(p2_hbm.at[idx.at[b, 1]], p2r[b], sem2[b]).wait()

    issue(0, 0)

    pltpu.sync_copy(wa2_hbm, wa2v)
    pltpu.sync_copy(ba2_hbm, ba2v)

    pltpu.sync_copy(aidx_hbm.at[pl.ds(w * APW, APW)], aidx)
    pltpu.async_copy(x_hbm.at[aidx], arows, sal).wait()
    pltpu.sync_copy(arows, arows_hbm.at[pl.ds(w * APW, APW)])

    lane = lax.iota(jnp.int32, 16)

    def compute(c, b):
        @pl.when(c >= 2)
        def _():
            pltpu.make_async_copy(qout[b], q_hbm.at[pl.ds(ebase, CH)],
                                  semq[b]).wait()

        def group(gi, _):
            e0 = gi * 16

            def edge(k, _):
                e = e0 + k
                acc = ba2v[...]
                for jj in range(8):
                    a = p1r[b][e, pl.ds(16 * jj, 16)]
                    bb = p2r[b][e, pl.ds(16 * jj, 16)]
                    acc = acc + jnp.maximum(a + bb, 0.0) * wa2v[jj]
                plsc.store_scatter(tbuf, [lane, lane * 0 + k], acc)
                return 0

            lax.fori_loop(0, 16, edge, 0)
            acc16 = tbuf[0]
            for r in range(1, 16):
                acc16 = acc16 + tbuf[r]
            qout[b][pl.ds(e0, 16)] = acc16
            return 0

        lax.fori_loop(0, CH // 16, group, 0)
        pltpu.async_copy(qout[b], q_hbm.at[pl.ds(ebase + c * CH, CH)],
                         semq[b])

    def step(i, _):
        issue(2 * i + 1, 1)
        wait(0)
        compute(2 * i, 0)

        @pl.when(i < NIT - 1)
        def _():
            issue(2 * i + 2, 0)

        wait(1)
        compute(2 * i + 1, 1)
        return 0

    lax.fori_loop(0, NIT, step, 0)
    pltpu.make_async_copy(qout0, q_hbm.at[pl.ds(ebase, CH)], sqa).wait()
    pltpu.make_async_copy(qout1, q_hbm.at[pl.ds(ebase, CH)], sqb).wait()


BR = 2000
NBLK = N // BR


def _k1_body(x_ref, w_ref, y_ref):
    y_ref[...] = jnp.maximum(_mm(x_ref[...], w_ref[...]), 0.0)


def _tc_relu_matmul(x, w):
    return pl.pallas_call(
        _k1_body,
        grid=(NBLK,),
        in_specs=[pl.BlockSpec((BR, D), lambda i: (i, 0)),
                  pl.BlockSpec((D, D), lambda i: (0, 0))],
        out_specs=pl.BlockSpec((BR, D), lambda i: (i, 0)),
        out_shape=jax.ShapeDtypeStruct((N, D), f32),
    )(x, w)


def _ln_relu(h, gam, bet):
    mu = jnp.mean(h, axis=-1, keepdims=True)
    var = jnp.mean((h - mu) ** 2, axis=-1, keepdims=True)
    xn = (h - mu) / jnp.sqrt(var + 1e-5) * gam + bet
    return jnp.maximum(xn, 0.0)


def _k2_body(x_ref, agg_ref, wn1, wn2, bnr, gam, bet, wm, x1_ref, y1_ref):
    agg = agg_ref[0] + agg_ref[1]
    h = _mm(x_ref[...], wn1[...]) + _mm(agg, wn2[...]) + bnr[...]
    x1 = _ln_relu(h, gam[...], bet[...])
    x1_ref[...] = x1
    y1_ref[...] = jnp.maximum(_mm(x1, wm[...]), 0.0)


def _tc_layer_fused(x, aggs, wn1, wn2, bnr, gam, bet, wm):
    return pl.pallas_call(
        _k2_body,
        grid=(NBLK,),
        in_specs=[pl.BlockSpec((BR, D), lambda i: (i, 0)),
                  pl.BlockSpec((NC, BR, D), lambda i: (0, i, 0)),
                  pl.BlockSpec((D, D), lambda i: (0, 0)),
                  pl.BlockSpec((D, D), lambda i: (0, 0)),
                  pl.BlockSpec((1, D), lambda i: (0, 0)),
                  pl.BlockSpec((1, D), lambda i: (0, 0)),
                  pl.BlockSpec((1, D), lambda i: (0, 0)),
                  pl.BlockSpec((D, D), lambda i: (0, 0))],
        out_specs=[pl.BlockSpec((BR, D), lambda i: (i, 0)),
                   pl.BlockSpec((BR, D), lambda i: (i, 0))],
        out_shape=[jax.ShapeDtypeStruct((N, D), f32),
                   jax.ShapeDtypeStruct((N, D), f32)],
    )(x, aggs, wn1, wn2, bnr, gam, bet, wm)


def _k3_body(x_ref, agg_ref, wn1, wn2, bnr, gam, bet, x2_ref, ps_ref):
    agg = agg_ref[0] + agg_ref[1]
    h = _mm(x_ref[...], wn1[...]) + _mm(agg, wn2[...]) + bnr[...]
    x2 = _ln_relu(h, gam[...], bet[...])
    x2_ref[...] = x2
    ps_ref[...] = jnp.sum(x2, axis=0, keepdims=True)[None]


def _tc_layer_final(x, aggs, wn1, wn2, bnr, gam, bet):
    return pl.pallas_call(
        _k3_body,
        grid=(NBLK,),
        in_specs=[pl.BlockSpec((BR, D), lambda i: (i, 0)),
                  pl.BlockSpec((NC, BR, D), lambda i: (0, i, 0)),
                  pl.BlockSpec((D, D), lambda i: (0, 0)),
                  pl.BlockSpec((D, D), lambda i: (0, 0)),
                  pl.BlockSpec((1, D), lambda i: (0, 0)),
                  pl.BlockSpec((1, D), lambda i: (0, 0)),
                  pl.BlockSpec((1, D), lambda i: (0, 0))],
        out_specs=[pl.BlockSpec((BR, D), lambda i: (i, 0)),
                   pl.BlockSpec((1, 1, D), lambda i: (i, 0, 0))],
        out_shape=[jax.ShapeDtypeStruct((N, D), f32),
                   jax.ShapeDtypeStruct((NBLK, 1, D), f32)],
    )(x, aggs, wn1, wn2, bnr, gam, bet)


def _k4_body(x_ref, ws, wd, c_ref, p1_ref, p2_ref):
    p1_ref[...] = _mm(x_ref[...], ws[...]) + c_ref[...]
    p2_ref[...] = _mm(x_ref[...], wd[...])


def _tc_p(x, ws, wd, c):
    return pl.pallas_call(
        _k4_body,
        grid=(NBLK,),
        in_specs=[pl.BlockSpec((BR, D), lambda i: (i, 0)),
                  pl.BlockSpec((D, D), lambda i: (0, 0)),
                  pl.BlockSpec((D, D), lambda i: (0, 0)),
                  pl.BlockSpec((1, D), lambda i: (0, 0))],
        out_specs=[pl.BlockSpec((BR, D), lambda i: (i, 0)),
                   pl.BlockSpec((BR, D), lambda i: (i, 0))],
        out_shape=[jax.ShapeDtypeStruct((N, D), f32),
                   jax.ShapeDtypeStruct((N, D), f32)],
    )(x, ws, wd, c)


def _k5_body(ax_ref, wmv1, cmv, wh1, chh, w2m, w2h, brow, out_ref):
    ax = ax_ref[...]
    hm = jnp.maximum(_mm(ax, wmv1[...]) + cmv[...], 0.0)
    hh = jnp.maximum(_mm(ax, wh1[...]) + chh[...], 0.0)
    out_ref[...] = _mm(hm, w2m[...]) + _mm(hh, w2h[...]) + brow[...]


def _tc_ally(arows, wmv1, cmv, wh1, chh, w2m, w2h, brow):
    return pl.pallas_call(
        _k5_body,
        grid=(1,),
        in_specs=[pl.BlockSpec((APAD, D), lambda i: (0, 0)),
                  pl.BlockSpec((D, D), lambda i: (0, 0)),
                  pl.BlockSpec((1, D), lambda i: (0, 0)),
                  pl.BlockSpec((D, D), lambda i: (0, 0)),
                  pl.BlockSpec((1, D), lambda i: (0, 0)),
                  pl.BlockSpec((D, D), lambda i: (0, 0)),
                  pl.BlockSpec((D, D), lambda i: (0, 0)),
                  pl.BlockSpec((1, D), lambda i: (0, 0))],
        out_specs=pl.BlockSpec((APAD, D), lambda i: (0, 0)),
        out_shape=jax.ShapeDtypeStruct((APAD, D), f32),
    )(arows, wmv1, cmv, wh1, chh, w2m, w2h, brow)


def kernel(node_feature, global_feature, Wm, Wn, bn, gamma, beta, Wg, bg,
           Wmv1, bmv1, Wmv2, bmv2, Wh1, bh1, Wh2, bh2, Wa1, ba1, Wa2, ba2,
           edge_index, attack_edge_index, ally_indices):
    adst = attack_edge_index[1]

    npad = EPAD - E
    zpad = jnp.zeros((npad,), jnp.int32)
    src_pad = jnp.concatenate([edge_index[0], zpad])
    dst_pad = jnp.concatenate(
        [edge_index[1], N + (jnp.arange(npad, dtype=jnp.int32) % (NP - N))])
    asrc_pad = jnp.concatenate([attack_edge_index[0], zpad])
    adst_pad = jnp.concatenate([attack_edge_index[1], zpad])

    y0 = _tc_relu_matmul(node_feature, Wm[0])
    aggs0 = _sc_segment_sum_kernel()(y0, src_pad, dst_pad)
    x1, y1 = _tc_layer_fused(node_feature, aggs0,
                             Wn[0][:D], Wn[0][D:], bn[0].reshape(1, D),
                             gamma[0].reshape(1, D), beta[0].reshape(1, D),
                             Wm[1])
    aggs1 = _sc_segment_sum_kernel()(y1, src_pad, dst_pad)
    x2, psum = _tc_layer_final(x1, aggs1,
                               Wn[1][:D], Wn[1][D:], bn[1].reshape(1, D),
                               gamma[1].reshape(1, D), beta[1].reshape(1, D))

    pooled = jnp.sum(psum[:, 0, :], axis=0, keepdims=True) / 10000.0
    g = jax.nn.relu(jnp.concatenate([global_feature, pooled], axis=-1) @ Wg + bg)

    c = g @ Wa1[D:D + G] + g @ Wa1[2 * D + G:] + ba1.reshape(1, D)
    P1, P2 = _tc_p(x2, Wa1[:D], Wa1[D + G:2 * D + G], c)

    aidx_pad = jnp.concatenate(
        [ally_indices, jnp.zeros((APAD - A,), jnp.int32)])
    wa2_r = Wa2.reshape(8, 16)
    ba2v = jnp.full((16,), ba2[0] / 16.0, f32)
    q_full, arows = _sc_attack_kernel()(P1, P2, asrc_pad, adst_pad,
                                        wa2_r, ba2v, x2, aidx_pad)
    q_attack = q_full[:E]

    cmv = g @ Wmv1[D:] + bmv1.reshape(1, H)
    chh = g @ Wh1[D:] + bh1.reshape(1, H)
    w2m = jnp.zeros((H, D), f32).at[:, :4].set(Wmv2)
    w2h = jnp.zeros((H, D), f32).at[:, 4:5].set(Wh2)
    brow = jnp.zeros((1, D), f32).at[0, :4].set(bmv2).at[0, 4].set(bh2[0])
    out5 = _tc_ally(arows, Wmv1[:D], cmv, Wh1[:D], chh, w2m, w2h, brow)

    q_move = out5[:A, :4]
    q_hold = out5[:A, 4]
    return q_move, q_hold, q_attack, adst


# speedup vs baseline: 1.7302x; 1.7302x over previous
import functools

import jax
import jax.numpy as jnp
from jax import lax
from jax.experimental import pallas as pl
from jax.experimental.pallas import tpu as pltpu
from jax.experimental.pallas import tpu_sc as plsc

N = 10000
E = 320000
D = 128
G = 32
H = 128
A = 2000

NC = 2
NS = 16
NW = NC * NS
CH = 80
EPW = 10240
EPAD = EPW * NW
NCHUNK = EPW // CH
NIT = NCHUNK // 2
NP = 10240
RPT = NP // NS
ZR = 128
NZ = RPT // ZR
APAD = 2048
APW = APAD // NW

_HI = jax.lax.Precision.HIGHEST
f32 = jnp.float32

@functools.lru_cache(maxsize=1)
def _sc_mesh():
    return plsc.VectorSubcoreMesh(core_axis_name="c", subcore_axis_name="s",
                                  num_cores=NC, num_subcores=NS)


def _mm(a, b):
    return jnp.matmul(a, b, precision=_HI)


@functools.lru_cache(maxsize=1)
def _sc_segment_sum_kernel():
    return pl.kernel(
        _sc_segment_sum_body,
        out_type=jax.ShapeDtypeStruct((NC, NP, D), f32),
        mesh=_sc_mesh(),
        compiler_params=pltpu.CompilerParams(needs_layout_passes=False),
        scratch_types=[
            pltpu.VMEM((2, 2, CH), jnp.int32),
            pltpu.VMEM((CH, D), f32),
            pltpu.VMEM((CH, D), f32),
            pltpu.VMEM((ZR, D), f32),
            pltpu.VMEM_SHARED((NP, D), f32),
            pltpu.SemaphoreType.DMA,
            pltpu.SemaphoreType.DMA,
        ],
    )


def _sc_segment_sum_body(y_hbm, src_hbm, dst_hbm, out_hbm,
                         idx, rows0, rows1, buf, acc, sem0, sem1):
    cid = lax.axis_index("c")
    sid = lax.axis_index("s")
    rows = (rows0, rows1)
    sems = (sem0, sem1)
    ebase = (cid * NS + sid) * EPW

    def issue(c, b):
        base = ebase + c * CH
        pltpu.sync_copy(src_hbm.at[pl.ds(base, CH)], idx.at[b, 0])
        pltpu.sync_copy(dst_hbm.at[pl.ds(base, CH)], idx.at[b, 1])
        pltpu.async_copy(y_hbm.at[idx.at[b, 0]], rows[b], sems[b])

    def wait(b):
        pltpu.make_async_copy(y_hbm.at[idx.at[b, 0]], rows[b], sems[b]).wait()

    def scatter(b):
        pltpu.sync_copy(rows[b], acc.at[idx.at[b, 1]], add=True)

    issue(0, 0)
    zv = jnp.zeros((16,), f32)

    def zrow(r, _):
        for jj in range(8):
            buf[r, pl.ds(16 * jj, 16)] = zv
        return 0

    lax.fori_loop(0, ZR, zrow, 0)
    for z in range(NZ):
        pltpu.sync_copy(buf, acc.at[pl.ds(sid * RPT + z * ZR, ZR)])
    plsc.subcore_barrier()

    def step(i, _):
        issue(2 * i + 1, 1)
        wait(0)
        scatter(0)

        @pl.when(i < NIT - 1)
        def _():
            issue(2 * i + 2, 0)

        wait(1)
        scatter(1)
        return 0

    lax.fori_loop(0, NIT, step, 0)
    plsc.subcore_barrier()

    for z in range(NZ):
        r0 = sid * RPT + z * ZR
        pltpu.sync_copy(acc.at[pl.ds(r0, ZR)],
                        out_hbm.at[cid, pl.ds(r0, ZR)])


@functools.lru_cache(maxsize=1)
def _sc_attack_kernel():
    return pl.kernel(
        _sc_attack_body,
        out_type=(
            jax.ShapeDtypeStruct((EPAD,), f32),
            jax.ShapeDtypeStruct((APAD, D), f32),
        ),
        mesh=_sc_mesh(),
        compiler_params=pltpu.CompilerParams(needs_layout_passes=False),
        scratch_types=[
            pltpu.VMEM((2, 2, CH), jnp.int32),
            pltpu.VMEM((CH, D), f32),
            pltpu.VMEM((CH, D), f32),
            pltpu.VMEM((CH, D), f32),
            pltpu.VMEM((CH, D), f32),
            pltpu.VMEM((CH,), f32),
            pltpu.VMEM((CH,), f32),
            pltpu.VMEM((8, 16), f32),
            pltpu.VMEM((16,), f32),
            pltpu.VMEM((APW,), jnp.int32),
            pltpu.VMEM((APW, D), f32),
            pltpu.VMEM((16, 16), f32),
            pltpu.SemaphoreType.DMA,
            pltpu.SemaphoreType.DMA,
            pltpu.SemaphoreType.DMA,
            pltpu.SemaphoreType.DMA,
            pltpu.SemaphoreType.DMA,
            pltpu.SemaphoreType.DMA,
            pltpu.SemaphoreType.DMA,
        ],
    )


def _sc_attack_body(p1_hbm, p2_hbm, asrc_hbm, adst_hbm, wa2_hbm, ba2_hbm,
                    x_hbm, aidx_hbm, q_hbm, arows_hbm,
                    idx, p1r0, p1r1, p2r0, p2r1, qout0, qout1,
                    wa2v, ba2v, aidx, arows, tbuf,
                    s1a, s1b, s2a, s2b, sqa, sqb, sal):
    cid = lax.axis_index("c")
    sid = lax.axis_index("s")
    w = cid * NS + sid
    ebase = w * EPW
    p1r = (p1r0, p1r1)
    p2r = (p2r0, p2r1)
    qout = (qout0, qout1)
    sem1 = (s1a, s1b)
    sem2 = (s2a, s2b)
    semq = (sqa, sqb)

    def issue(c, b):
        base = ebase + c * CH
        pltpu.sync_copy(asrc_hbm.at[pl.ds(base, CH)], idx.at[b, 0])
        pltpu.sync_copy(adst_hbm.at[pl.ds(base, CH)], idx.at[b, 1])
        pltpu.async_copy(p1_hbm.at[idx.at[b, 0]], p1r[b], sem1[b])
        pltpu.async_copy(p2_hbm.at[idx.at[b, 1]], p2r[b], sem2[b])

    def wait(b):
        pltpu.make_async_copy(p1_hbm.at[idx.at[b, 0]], p1r[b], sem1[b]).wait()
        pltpu.make_async_copy(p2_hbm.at[idx.at[b, 1]], p2r[b], sem2[b]).wait()

    issue(0, 0)

    pltpu.sync_copy(wa2_hbm, wa2v)
    pltpu.sync_copy(ba2_hbm, ba2v)

    pltpu.sync_copy(aidx_hbm.at[pl.ds(w * APW, APW)], aidx)
    pltpu.async_copy(x_hbm.at[aidx], arows, sal).wait()
    pltpu.sync_copy(arows, arows_hbm.at[pl.ds(w * APW, APW)])

    lane = lax.iota(jnp.int32, 16)

    def compute(c, b):
        @pl.when(c >= 2)
        def _():
            pltpu.make_async_copy(qout[b], q_hbm.at[pl.ds(ebase, CH)],
                                  semq[b]).wait()

        def group(gi, _):
            e0 = gi * 16

            def edge(k, _):
                e = e0 + k
                acc = ba2v[...]
                for jj in range(8):
                    a = p1r[b][e, pl.ds(16 * jj, 16)]
                    bb = p2r[b][e, pl.ds(16 * jj, 16)]
                    acc = acc + jnp.maximum(a + bb, 0.0) * wa2v[jj]
                plsc.store_scatter(tbuf, [lane, lane * 0 + k], acc)
                return 0

            lax.fori_loop(0, 16, edge, 0)
            acc16 = tbuf[0]
            for r in range(1, 16):
                acc16 = acc16 + tbuf[r]
            qout[b][pl.ds(e0, 16)] = acc16
            return 0

        lax.fori_loop(0, CH // 16, group, 0)
        pltpu.async_copy(qout[b], q_hbm.at[pl.ds(ebase + c * CH, CH)],
                         semq[b])

    def step(i, _):
        issue(2 * i + 1, 1)
        wait(0)
        compute(2 * i, 0)

        @pl.when(i < NIT - 1)
        def _():
            issue(2 * i + 2, 0)

        wait(1)
        compute(2 * i + 1, 1)
        return 0

    lax.fori_loop(0, NIT, step, 0)
    pltpu.make_async_copy(qout0, q_hbm.at[pl.ds(ebase, CH)], sqa).wait()
    pltpu.make_async_copy(qout1, q_hbm.at[pl.ds(ebase, CH)], sqb).wait()


BR = 2000
NBLK = N // BR


def _k1_body(x_ref, w_ref, y_ref):
    y_ref[...] = jnp.maximum(_mm(x_ref[...], w_ref[...]), 0.0)


def _tc_relu_matmul(x, w):
    return pl.pallas_call(
        _k1_body,
        grid=(NBLK,),
        in_specs=[pl.BlockSpec((BR, D), lambda i: (i, 0)),
                  pl.BlockSpec((D, D), lambda i: (0, 0))],
        out_specs=pl.BlockSpec((BR, D), lambda i: (i, 0)),
        out_shape=jax.ShapeDtypeStruct((N, D), f32),
    )(x, w)


def _ln_relu(h, gam, bet):
    mu = jnp.mean(h, axis=-1, keepdims=True)
    var = jnp.mean((h - mu) ** 2, axis=-1, keepdims=True)
    xn = (h - mu) / jnp.sqrt(var + 1e-5) * gam + bet
    return jnp.maximum(xn, 0.0)


def _k2_body(x_ref, agg_ref, wn1, wn2, bnr, gam, bet, wm, x1_ref, y1_ref):
    agg = agg_ref[0] + agg_ref[1]
    h = _mm(x_ref[...], wn1[...]) + _mm(agg, wn2[...]) + bnr[...]
    x1 = _ln_relu(h, gam[...], bet[...])
    x1_ref[...] = x1
    y1_ref[...] = jnp.maximum(_mm(x1, wm[...]), 0.0)


def _tc_layer_fused(x, aggs, wn1, wn2, bnr, gam, bet, wm):
    return pl.pallas_call(
        _k2_body,
        grid=(NBLK,),
        in_specs=[pl.BlockSpec((BR, D), lambda i: (i, 0)),
                  pl.BlockSpec((NC, BR, D), lambda i: (0, i, 0)),
                  pl.BlockSpec((D, D), lambda i: (0, 0)),
                  pl.BlockSpec((D, D), lambda i: (0, 0)),
                  pl.BlockSpec((1, D), lambda i: (0, 0)),
                  pl.BlockSpec((1, D), lambda i: (0, 0)),
                  pl.BlockSpec((1, D), lambda i: (0, 0)),
                  pl.BlockSpec((D, D), lambda i: (0, 0))],
        out_specs=[pl.BlockSpec((BR, D), lambda i: (i, 0)),
                   pl.BlockSpec((BR, D), lambda i: (i, 0))],
        out_shape=[jax.ShapeDtypeStruct((N, D), f32),
                   jax.ShapeDtypeStruct((N, D), f32)],
    )(x, aggs, wn1, wn2, bnr, gam, bet, wm)


def _k3_body(x_ref, agg_ref, wn1, wn2, bnr, gam, bet, x2_ref, ps_ref):
    agg = agg_ref[0] + agg_ref[1]
    h = _mm(x_ref[...], wn1[...]) + _mm(agg, wn2[...]) + bnr[...]
    x2 = _ln_relu(h, gam[...], bet[...])
    x2_ref[...] = x2
    ps_ref[...] = jnp.sum(x2, axis=0, keepdims=True)[None]


def _tc_layer_final(x, aggs, wn1, wn2, bnr, gam, bet):
    return pl.pallas_call(
        _k3_body,
        grid=(NBLK,),
        in_specs=[pl.BlockSpec((BR, D), lambda i: (i, 0)),
                  pl.BlockSpec((NC, BR, D), lambda i: (0, i, 0)),
                  pl.BlockSpec((D, D), lambda i: (0, 0)),
                  pl.BlockSpec((D, D), lambda i: (0, 0)),
                  pl.BlockSpec((1, D), lambda i: (0, 0)),
                  pl.BlockSpec((1, D), lambda i: (0, 0)),
                  pl.BlockSpec((1, D), lambda i: (0, 0))],
        out_specs=[pl.BlockSpec((BR, D), lambda i: (i, 0)),
                   pl.BlockSpec((1, 1, D), lambda i: (i, 0, 0))],
        out_shape=[jax.ShapeDtypeStruct((N, D), f32),
                   jax.ShapeDtypeStruct((NBLK, 1, D), f32)],
    )(x, aggs, wn1, wn2, bnr, gam, bet)


def _k4_body(x_ref, ws, wd, c_ref, p1_ref, p2_ref):
    p1_ref[...] = _mm(x_ref[...], ws[...]) + c_ref[...]
    p2_ref[...] = _mm(x_ref[...], wd[...])


def _tc_p(x, ws, wd, c):
    return pl.pallas_call(
        _k4_body,
        grid=(NBLK,),
        in_specs=[pl.BlockSpec((BR, D), lambda i: (i, 0)),
                  pl.BlockSpec((D, D), lambda i: (0, 0)),
                  pl.BlockSpec((D, D), lambda i: (0, 0)),
                  pl.BlockSpec((1, D), lambda i: (0, 0))],
        out_specs=[pl.BlockSpec((BR, D), lambda i: (i, 0)),
                   pl.BlockSpec((BR, D), lambda i: (i, 0))],
        out_shape=[jax.ShapeDtypeStruct((N, D), f32),
                   jax.ShapeDtypeStruct((N, D), f32)],
    )(x, ws, wd, c)


def _k5_body(ax_ref, wmv1, cmv, wh1, chh, w2m, w2h, brow, out_ref):
    ax = ax_ref[...]
    hm = jnp.maximum(_mm(ax, wmv1[...]) + cmv[...], 0.0)
    hh = jnp.maximum(_mm(ax, wh1[...]) + chh[...], 0.0)
    out_ref[...] = _mm(hm, w2m[...]) + _mm(hh, w2h[...]) + brow[...]


def _tc_ally(arows, wmv1, cmv, wh1, chh, w2m, w2h, brow):
    return pl.pallas_call(
        _k5_body,
        grid=(1,),
        in_specs=[pl.BlockSpec((APAD, D), lambda i: (0, 0)),
                  pl.BlockSpec((D, D), lambda i: (0, 0)),
                  pl.BlockSpec((1, D), lambda i: (0, 0)),
                  pl.BlockSpec((D, D), lambda i: (0, 0)),
                  pl.BlockSpec((1, D), lambda i: (0, 0)),
                  pl.BlockSpec((D, D), lambda i: (0, 0)),
                  pl.BlockSpec((D, D), lambda i: (0, 0)),
                  pl.BlockSpec((1, D), lambda i: (0, 0))],
        out_specs=pl.BlockSpec((APAD, D), lambda i: (0, 0)),
        out_shape=jax.ShapeDtypeStruct((APAD, D), f32),
    )(arows, wmv1, cmv, wh1, chh, w2m, w2h, brow)


def kernel(node_feature, global_feature, Wm, Wn, bn, gamma, beta, Wg, bg,
           Wmv1, bmv1, Wmv2, bmv2, Wh1, bh1, Wh2, bh2, Wa1, ba1, Wa2, ba2,
           edge_index, attack_edge_index, ally_indices):
    adst = attack_edge_index[1]

    npad = EPAD - E
    spread = jnp.arange(npad, dtype=jnp.int32) * 13 % N
    src_pad = jnp.concatenate([edge_index[0], spread])
    dst_pad = jnp.concatenate(
        [edge_index[1], N + (jnp.arange(npad, dtype=jnp.int32) % (NP - N))])
    asrc_pad = jnp.concatenate([attack_edge_index[0], spread])
    adst_pad = jnp.concatenate([attack_edge_index[1], spread])

    y0 = _tc_relu_matmul(node_feature, Wm[0])
    aggs0 = _sc_segment_sum_kernel()(y0, src_pad, dst_pad)
    x1, y1 = _tc_layer_fused(node_feature, aggs0,
                             Wn[0][:D], Wn[0][D:], bn[0].reshape(1, D),
                             gamma[0].reshape(1, D), beta[0].reshape(1, D),
                             Wm[1])
    aggs1 = _sc_segment_sum_kernel()(y1, src_pad, dst_pad)
    x2, psum = _tc_layer_final(x1, aggs1,
                               Wn[1][:D], Wn[1][D:], bn[1].reshape(1, D),
                               gamma[1].reshape(1, D), beta[1].reshape(1, D))

    pooled = jnp.sum(psum[:, 0, :], axis=0, keepdims=True) / 10000.0
    g = jax.nn.relu(jnp.concatenate([global_feature, pooled], axis=-1) @ Wg + bg)

    c = g @ Wa1[D:D + G] + g @ Wa1[2 * D + G:] + ba1.reshape(1, D)
    P1, P2 = _tc_p(x2, Wa1[:D], Wa1[D + G:2 * D + G], c)

    aidx_pad = jnp.concatenate(
        [ally_indices, jnp.zeros((APAD - A,), jnp.int32)])
    wa2_r = Wa2.reshape(8, 16)
    ba2v = jnp.full((16,), ba2[0] / 16.0, f32)
    q_full, arows = _sc_attack_kernel()(P1, P2, asrc_pad, adst_pad,
                                        wa2_r, ba2v, x2, aidx_pad)
    q_attack = q_full[:E]

    cmv = g @ Wmv1[D:] + bmv1.reshape(1, H)
    chh = g @ Wh1[D:] + bh1.reshape(1, H)
    w2m = jnp.zeros((H, D), f32).at[:, :4].set(Wmv2)
    w2h = jnp.zeros((H, D), f32).at[:, 4:5].set(Wh2)
    brow = jnp.zeros((1, D), f32).at[0, :4].set(bmv2).at[0, 4].set(bh2[0])
    out5 = _tc_ally(arows, Wmv1[:D], cmv, Wh1[:D], chh, w2m, w2h, brow)

    q_move = out5[:A, :4]
    q_hold = out5[:A, 4]
    return q_move, q_hold, q_attack, adst


# speedup vs baseline: 1.7336x; 1.0020x over previous
import functools

import jax
import jax.numpy as jnp
from jax import lax
from jax.experimental import pallas as pl
from jax.experimental.pallas import tpu as pltpu
from jax.experimental.pallas import tpu_sc as plsc

N = 10000
E = 320000
D = 128
G = 32
H = 128
A = 2000

NC = 2
NS = 16
NW = NC * NS
CH = 80
EPW = 10240
EPAD = EPW * NW
NCHUNK = EPW // CH
NIT = NCHUNK // 2
NP = 10240
RPT = NP // NS
ZR = 128
NZ = RPT // ZR
APAD = 2048
APW = APAD // NW

_HI = jax.lax.Precision.HIGHEST
f32 = jnp.float32

@functools.lru_cache(maxsize=1)
def _sc_mesh():
    return plsc.VectorSubcoreMesh(core_axis_name="c", subcore_axis_name="s",
                                  num_cores=NC, num_subcores=NS)


def _mm(a, b):
    return jnp.matmul(a.astype(jnp.bfloat16), b.astype(jnp.bfloat16),
                      preferred_element_type=f32)


@functools.lru_cache(maxsize=1)
def _sc_segment_sum_kernel():
    return pl.kernel(
        _sc_segment_sum_body,
        out_type=jax.ShapeDtypeStruct((NC, NP, D), f32),
        mesh=_sc_mesh(),
        compiler_params=pltpu.CompilerParams(needs_layout_passes=False),
        scratch_types=[
            pltpu.VMEM((2, 2, CH), jnp.int32),
            pltpu.VMEM((CH, D), f32),
            pltpu.VMEM((CH, D), f32),
            pltpu.VMEM((ZR, D), f32),
            pltpu.VMEM_SHARED((NP, D), f32),
            pltpu.SemaphoreType.DMA,
            pltpu.SemaphoreType.DMA,
        ],
    )


def _sc_segment_sum_body(y_hbm, src_hbm, dst_hbm, out_hbm,
                         idx, rows0, rows1, buf, acc, sem0, sem1):
    cid = lax.axis_index("c")
    sid = lax.axis_index("s")
    rows = (rows0, rows1)
    sems = (sem0, sem1)
    ebase = (cid * NS + sid) * EPW

    def issue(c, b):
        base = ebase + c * CH
        pltpu.sync_copy(src_hbm.at[pl.ds(base, CH)], idx.at[b, 0])
        pltpu.sync_copy(dst_hbm.at[pl.ds(base, CH)], idx.at[b, 1])
        pltpu.async_copy(y_hbm.at[idx.at[b, 0]], rows[b], sems[b])

    def wait(b):
        pltpu.make_async_copy(y_hbm.at[idx.at[b, 0]], rows[b], sems[b]).wait()

    def scatter(b):
        pltpu.sync_copy(rows[b], acc.at[idx.at[b, 1]], add=True)

    issue(0, 0)
    zv = jnp.zeros((16,), f32)

    def zrow(r, _):
        for jj in range(8):
            buf[r, pl.ds(16 * jj, 16)] = zv
        return 0

    lax.fori_loop(0, ZR, zrow, 0)
    for z in range(NZ):
        pltpu.sync_copy(buf, acc.at[pl.ds(sid * RPT + z * ZR, ZR)])
    plsc.subcore_barrier()

    def step(i, _):
        issue(2 * i + 1, 1)
        wait(0)
        scatter(0)

        @pl.when(i < NIT - 1)
        def _():
            issue(2 * i + 2, 0)

        wait(1)
        scatter(1)
        return 0

    lax.fori_loop(0, NIT, step, 0)
    plsc.subcore_barrier()

    for z in range(NZ):
        r0 = sid * RPT + z * ZR
        pltpu.sync_copy(acc.at[pl.ds(r0, ZR)],
                        out_hbm.at[cid, pl.ds(r0, ZR)])


@functools.lru_cache(maxsize=1)
def _sc_attack_kernel():
    return pl.kernel(
        _sc_attack_body,
        out_type=(
            jax.ShapeDtypeStruct((EPAD,), f32),
            jax.ShapeDtypeStruct((APAD, D), f32),
        ),
        mesh=_sc_mesh(),
        compiler_params=pltpu.CompilerParams(needs_layout_passes=False),
        scratch_types=[
            pltpu.VMEM((2, 2, CH), jnp.int32),
            pltpu.VMEM((CH, D), f32),
            pltpu.VMEM((CH, D), f32),
            pltpu.VMEM((CH, D), f32),
            pltpu.VMEM((CH, D), f32),
            pltpu.VMEM((CH,), f32),
            pltpu.VMEM((CH,), f32),
            pltpu.VMEM((8, 16), f32),
            pltpu.VMEM((16,), f32),
            pltpu.VMEM((APW,), jnp.int32),
            pltpu.VMEM((APW, D), f32),
            pltpu.VMEM((16, 16), f32),
            pltpu.SemaphoreType.DMA,
            pltpu.SemaphoreType.DMA,
            pltpu.SemaphoreType.DMA,
            pltpu.SemaphoreType.DMA,
            pltpu.SemaphoreType.DMA,
            pltpu.SemaphoreType.DMA,
            pltpu.SemaphoreType.DMA,
        ],
    )


def _sc_attack_body(p1_hbm, p2_hbm, asrc_hbm, adst_hbm, wa2_hbm, ba2_hbm,
                    x_hbm, aidx_hbm, q_hbm, arows_hbm,
                    idx, p1r0, p1r1, p2r0, p2r1, qout0, qout1,
                    wa2v, ba2v, aidx, arows, tbuf,
                    s1a, s1b, s2a, s2b, sqa, sqb, sal):
    cid = lax.axis_index("c")
    sid = lax.axis_index("s")
    w = cid * NS + sid
    ebase = w * EPW
    p1r = (p1r0, p1r1)
    p2r = (p2r0, p2r1)
    qout = (qout0, qout1)
    sem1 = (s1a, s1b)
    sem2 = (s2a, s2b)
    semq = (sqa, sqb)

    def issue(c, b):
        base = ebase + c * CH
        pltpu.sync_copy(asrc_hbm.at[pl.ds(base, CH)], idx.at[b, 0])
        pltpu.sync_copy(adst_hbm.at[pl.ds(base, CH)], idx.at[b, 1])
        pltpu.async_copy(p1_hbm.at[idx.at[b, 0]], p1r[b], sem1[b])
        pltpu.async_copy(p2_hbm.at[idx.at[b, 1]], p2r[b], sem2[b])

    def wait(b):
        pltpu.make_async_copy(p1_hbm.at[idx.at[b, 0]], p1r[b], sem1[b]).wait()
        pltpu.make_async_copy(p2_hbm.at[idx.at[b, 1]], p2r[b], sem2[b]).wait()

    issue(0, 0)

    pltpu.sync_copy(wa2_hbm, wa2v)
    pltpu.sync_copy(ba2_hbm, ba2v)

    pltpu.sync_copy(aidx_hbm.at[pl.ds(w * APW, APW)], aidx)
    pltpu.async_copy(x_hbm.at[aidx], arows, sal).wait()
    pltpu.sync_copy(arows, arows_hbm.at[pl.ds(w * APW, APW)])

    lane = lax.iota(jnp.int32, 16)

    def compute(c, b):
        @pl.when(c >= 2)
        def _():
            pltpu.make_async_copy(qout[b], q_hbm.at[pl.ds(ebase, CH)],
                                  semq[b]).wait()

        def group(gi, _):
            e0 = gi * 16

            def edge(k, _):
                e = e0 + k
                acc = ba2v[...]
                for jj in range(8):
                    a = p1r[b][e, pl.ds(16 * jj, 16)]
                    bb = p2r[b][e, pl.ds(16 * jj, 16)]
                    t = jnp.maximum(a + bb, 0.0)
                    u = plsc.bitcast(t, jnp.uint32)
                    rnd = (u + jnp.uint32(0x7FFF)
                           + (lax.shift_right_logical(u, jnp.uint32(16))
                              & jnp.uint32(1))) & jnp.uint32(0xFFFF0000)
                    t = plsc.bitcast(rnd, f32)
                    acc = acc + t * wa2v[jj]
                plsc.store_scatter(tbuf, [lane, lane * 0 + k], acc)
                return 0

            lax.fori_loop(0, 16, edge, 0)
            acc16 = tbuf[0]
            for r in range(1, 16):
                acc16 = acc16 + tbuf[r]
            qout[b][pl.ds(e0, 16)] = acc16
            return 0

        lax.fori_loop(0, CH // 16, group, 0)
        pltpu.async_copy(qout[b], q_hbm.at[pl.ds(ebase + c * CH, CH)],
                         semq[b])

    def step(i, _):
        issue(2 * i + 1, 1)
        wait(0)
        compute(2 * i, 0)

        @pl.when(i < NIT - 1)
        def _():
            issue(2 * i + 2, 0)

        wait(1)
        compute(2 * i + 1, 1)
        return 0

    lax.fori_loop(0, NIT, step, 0)
    pltpu.make_async_copy(qout0, q_hbm.at[pl.ds(ebase, CH)], sqa).wait()
    pltpu.make_async_copy(qout1, q_hbm.at[pl.ds(ebase, CH)], sqb).wait()


BR = 2000
NBLK = N // BR


def _k1_body(x_ref, w_ref, y_ref):
    y_ref[...] = jnp.maximum(_mm(x_ref[...], w_ref[...]), 0.0)


def _tc_relu_matmul(x, w):
    return pl.pallas_call(
        _k1_body,
        grid=(NBLK,),
        in_specs=[pl.BlockSpec((BR, D), lambda i: (i, 0)),
                  pl.BlockSpec((D, D), lambda i: (0, 0))],
        out_specs=pl.BlockSpec((BR, D), lambda i: (i, 0)),
        out_shape=jax.ShapeDtypeStruct((N, D), f32),
    )(x, w)


def _ln_relu(h, gam, bet):
    mu = jnp.mean(h, axis=-1, keepdims=True)
    var = jnp.mean((h - mu) ** 2, axis=-1, keepdims=True)
    xn = (h - mu) / jnp.sqrt(var + 1e-5) * gam + bet
    return jnp.maximum(xn, 0.0)


def _k2_body(x_ref, agg_ref, wn1, wn2, bnr, gam, bet, wm, x1_ref, y1_ref):
    agg = agg_ref[0] + agg_ref[1]
    h = _mm(x_ref[...], wn1[...]) + _mm(agg, wn2[...]) + bnr[...]
    x1 = _ln_relu(h, gam[...], bet[...])
    x1_ref[...] = x1
    y1_ref[...] = jnp.maximum(_mm(x1, wm[...]), 0.0)


def _tc_layer_fused(x, aggs, wn1, wn2, bnr, gam, bet, wm):
    return pl.pallas_call(
        _k2_body,
        grid=(NBLK,),
        in_specs=[pl.BlockSpec((BR, D), lambda i: (i, 0)),
                  pl.BlockSpec((NC, BR, D), lambda i: (0, i, 0)),
                  pl.BlockSpec((D, D), lambda i: (0, 0)),
                  pl.BlockSpec((D, D), lambda i: (0, 0)),
                  pl.BlockSpec((1, D), lambda i: (0, 0)),
                  pl.BlockSpec((1, D), lambda i: (0, 0)),
                  pl.BlockSpec((1, D), lambda i: (0, 0)),
                  pl.BlockSpec((D, D), lambda i: (0, 0))],
        out_specs=[pl.BlockSpec((BR, D), lambda i: (i, 0)),
                   pl.BlockSpec((BR, D), lambda i: (i, 0))],
        out_shape=[jax.ShapeDtypeStruct((N, D), f32),
                   jax.ShapeDtypeStruct((N, D), f32)],
    )(x, aggs, wn1, wn2, bnr, gam, bet, wm)


def _k3_body(x_ref, agg_ref, wn1, wn2, bnr, gam, bet, x2_ref, ps_ref):
    agg = agg_ref[0] + agg_ref[1]
    h = _mm(x_ref[...], wn1[...]) + _mm(agg, wn2[...]) + bnr[...]
    x2 = _ln_relu(h, gam[...], bet[...])
    x2_ref[...] = x2
    ps_ref[...] = jnp.sum(x2, axis=0, keepdims=True)[None]


def _tc_layer_final(x, aggs, wn1, wn2, bnr, gam, bet):
    return pl.pallas_call(
        _k3_body,
        grid=(NBLK,),
        in_specs=[pl.BlockSpec((BR, D), lambda i: (i, 0)),
                  pl.BlockSpec((NC, BR, D), lambda i: (0, i, 0)),
                  pl.BlockSpec((D, D), lambda i: (0, 0)),
                  pl.BlockSpec((D, D), lambda i: (0, 0)),
                  pl.BlockSpec((1, D), lambda i: (0, 0)),
                  pl.BlockSpec((1, D), lambda i: (0, 0)),
                  pl.BlockSpec((1, D), lambda i: (0, 0))],
        out_specs=[pl.BlockSpec((BR, D), lambda i: (i, 0)),
                   pl.BlockSpec((1, 1, D), lambda i: (i, 0, 0))],
        out_shape=[jax.ShapeDtypeStruct((N, D), f32),
                   jax.ShapeDtypeStruct((NBLK, 1, D), f32)],
    )(x, aggs, wn1, wn2, bnr, gam, bet)


def _k4_body(x_ref, ws, wd, c_ref, p1_ref, p2_ref):
    p1_ref[...] = _mm(x_ref[...], ws[...]) + c_ref[...]
    p2_ref[...] = _mm(x_ref[...], wd[...])


def _tc_p(x, ws, wd, c):
    return pl.pallas_call(
        _k4_body,
        grid=(NBLK,),
        in_specs=[pl.BlockSpec((BR, D), lambda i: (i, 0)),
                  pl.BlockSpec((D, D), lambda i: (0, 0)),
                  pl.BlockSpec((D, D), lambda i: (0, 0)),
                  pl.BlockSpec((1, D), lambda i: (0, 0))],
        out_specs=[pl.BlockSpec((BR, D), lambda i: (i, 0)),
                   pl.BlockSpec((BR, D), lambda i: (i, 0))],
        out_shape=[jax.ShapeDtypeStruct((N, D), f32),
                   jax.ShapeDtypeStruct((N, D), f32)],
    )(x, ws, wd, c)


def _k5_body(ax_ref, wmv1, cmv, wh1, chh, w2m, w2h, brow, out_ref):
    ax = ax_ref[...]
    hm = jnp.maximum(_mm(ax, wmv1[...]) + cmv[...], 0.0)
    hh = jnp.maximum(_mm(ax, wh1[...]) + chh[...], 0.0)
    out_ref[...] = _mm(hm, w2m[...]) + _mm(hh, w2h[...]) + brow[...]


def _tc_ally(arows, wmv1, cmv, wh1, chh, w2m, w2h, brow):
    return pl.pallas_call(
        _k5_body,
        grid=(1,),
        in_specs=[pl.BlockSpec((APAD, D), lambda i: (0, 0)),
                  pl.BlockSpec((D, D), lambda i: (0, 0)),
                  pl.BlockSpec((1, D), lambda i: (0, 0)),
                  pl.BlockSpec((D, D), lambda i: (0, 0)),
                  pl.BlockSpec((1, D), lambda i: (0, 0)),
                  pl.BlockSpec((D, D), lambda i: (0, 0)),
                  pl.BlockSpec((D, D), lambda i: (0, 0)),
                  pl.BlockSpec((1, D), lambda i: (0, 0))],
        out_specs=pl.BlockSpec((APAD, D), lambda i: (0, 0)),
        out_shape=jax.ShapeDtypeStruct((APAD, D), f32),
    )(arows, wmv1, cmv, wh1, chh, w2m, w2h, brow)


def kernel(node_feature, global_feature, Wm, Wn, bn, gamma, beta, Wg, bg,
           Wmv1, bmv1, Wmv2, bmv2, Wh1, bh1, Wh2, bh2, Wa1, ba1, Wa2, ba2,
           edge_index, attack_edge_index, ally_indices):
    adst = attack_edge_index[1]

    npad = EPAD - E
    spread = jnp.arange(npad, dtype=jnp.int32) * 13 % N
    src_pad = jnp.concatenate([edge_index[0], spread])
    dst_pad = jnp.concatenate(
        [edge_index[1], N + (jnp.arange(npad, dtype=jnp.int32) % (NP - N))])
    asrc_pad = jnp.concatenate([attack_edge_index[0], spread])
    adst_pad = jnp.concatenate([attack_edge_index[1], spread])

    y0 = _tc_relu_matmul(node_feature, Wm[0])
    aggs0 = _sc_segment_sum_kernel()(y0, src_pad, dst_pad)
    x1, y1 = _tc_layer_fused(node_feature, aggs0,
                             Wn[0][:D], Wn[0][D:], bn[0].reshape(1, D),
                             gamma[0].reshape(1, D), beta[0].reshape(1, D),
                             Wm[1])
    aggs1 = _sc_segment_sum_kernel()(y1, src_pad, dst_pad)
    x2, psum = _tc_layer_final(x1, aggs1,
                               Wn[1][:D], Wn[1][D:], bn[1].reshape(1, D),
                               gamma[1].reshape(1, D), beta[1].reshape(1, D))

    pooled = jnp.sum(psum[:, 0, :], axis=0, keepdims=True) / 10000.0
    g = jax.nn.relu(
        _mm(jnp.concatenate([global_feature, pooled], axis=-1), Wg) + bg)

    c = _mm(g, Wa1[D:D + G]) + _mm(g, Wa1[2 * D + G:]) + ba1.reshape(1, D)
    P1, P2 = _tc_p(x2, Wa1[:D], Wa1[D + G:2 * D + G], c)

    aidx_pad = jnp.concatenate(
        [ally_indices, jnp.zeros((APAD - A,), jnp.int32)])
    wa2_r = Wa2.astype(jnp.bfloat16).astype(f32).reshape(8, 16)
    ba2v = jnp.full((16,), ba2[0] / 16.0, f32)
    q_full, arows = _sc_attack_kernel()(P1, P2, asrc_pad, adst_pad,
                                        wa2_r, ba2v, x2, aidx_pad)
    q_attack = q_full[:E]

    cmv = _mm(g, Wmv1[D:]) + bmv1.reshape(1, H)
    chh = _mm(g, Wh1[D:]) + bh1.reshape(1, H)
    w2m = jnp.zeros((H, D), f32).at[:, :4].set(Wmv2)
    w2h = jnp.zeros((H, D), f32).at[:, 4:5].set(Wh2)
    brow = jnp.zeros((1, D), f32).at[0, :4].set(bmv2).at[0, 4].set(bh2[0])
    out5 = _tc_ally(arows, Wmv1[:D], cmv, Wh1[:D], chh, w2m, w2h, brow)

    q_move = out5[:A, :4]
    q_hold = out5[:A, 4]
    return q_move, q_hold, q_attack, adst


# speedup vs baseline: 1.7715x; 1.0218x over previous
import functools

import jax
import jax.numpy as jnp
from jax import lax
from jax.experimental import pallas as pl
from jax.experimental.pallas import tpu as pltpu
from jax.experimental.pallas import tpu_sc as plsc

N = 10000
E = 320000
D = 128
G = 32
H = 128
A = 2000

NC = 2
NS = 16
NW = NC * NS
CH = 80
EPW = 10240
EPAD = EPW * NW
NCHUNK = EPW // CH
NIT = NCHUNK // 2
NP = 10240
RPT = NP // NS
ZR = 128
NZ = RPT // ZR
APAD = 2048
APW = APAD // NW

_HI = jax.lax.Precision.HIGHEST
f32 = jnp.float32

@functools.lru_cache(maxsize=1)
def _sc_mesh():
    return plsc.VectorSubcoreMesh(core_axis_name="c", subcore_axis_name="s",
                                  num_cores=NC, num_subcores=NS)


def _mm(a, b):
    return jnp.matmul(a.astype(jnp.bfloat16), b.astype(jnp.bfloat16),
                      preferred_element_type=f32)


@functools.lru_cache(maxsize=1)
def _sc_segment_sum_kernel():
    return pl.kernel(
        _sc_segment_sum_body,
        out_type=jax.ShapeDtypeStruct((NC, NP, D), f32),
        mesh=_sc_mesh(),
        compiler_params=pltpu.CompilerParams(needs_layout_passes=False),
        scratch_types=[
            pltpu.VMEM((2, 2, CH), jnp.int32),
            pltpu.VMEM((CH, D), f32),
            pltpu.VMEM((CH, D), f32),
            pltpu.VMEM((ZR, D), f32),
            pltpu.VMEM_SHARED((NP, D), f32),
            pltpu.SemaphoreType.DMA,
            pltpu.SemaphoreType.DMA,
        ],
    )


def _sc_segment_sum_body(y_hbm, src_hbm, dst_hbm, out_hbm,
                         idx, rows0, rows1, buf, acc, sem0, sem1):
    cid = lax.axis_index("c")
    sid = lax.axis_index("s")
    rows = (rows0, rows1)
    sems = (sem0, sem1)
    ebase = (cid * NS + sid) * EPW

    def issue(c, b):
        base = ebase + c * CH
        pltpu.sync_copy(src_hbm.at[pl.ds(base, CH)], idx.at[b, 0])
        pltpu.sync_copy(dst_hbm.at[pl.ds(base, CH)], idx.at[b, 1])
        pltpu.async_copy(y_hbm.at[idx.at[b, 0]], rows[b], sems[b])

    def wait(b):
        pltpu.make_async_copy(y_hbm.at[idx.at[b, 0]], rows[b], sems[b]).wait()

    def scatter(b):
        pltpu.sync_copy(rows[b], acc.at[idx.at[b, 1]], add=True)

    issue(0, 0)
    zv = jnp.zeros((16,), f32)

    def zrow(r, _):
        for jj in range(8):
            buf[r, pl.ds(16 * jj, 16)] = zv
        return 0

    lax.fori_loop(0, ZR, zrow, 0)
    for z in range(NZ):
        pltpu.sync_copy(buf, acc.at[pl.ds(sid * RPT + z * ZR, ZR)])
    plsc.subcore_barrier()

    def step(i, _):
        issue(2 * i + 1, 1)
        wait(0)
        scatter(0)

        @pl.when(i < NIT - 1)
        def _():
            issue(2 * i + 2, 0)

        wait(1)
        scatter(1)
        return 0

    lax.fori_loop(0, NIT, step, 0)
    plsc.subcore_barrier()

    for z in range(NZ):
        r0 = sid * RPT + z * ZR
        pltpu.sync_copy(acc.at[pl.ds(r0, ZR)],
                        out_hbm.at[cid, pl.ds(r0, ZR)])


@functools.lru_cache(maxsize=1)
def _sc_attack_kernel():
    return pl.kernel(
        _sc_attack_body,
        out_type=(
            jax.ShapeDtypeStruct((EPAD,), f32),
            jax.ShapeDtypeStruct((APAD, D), f32),
        ),
        mesh=_sc_mesh(),
        compiler_params=pltpu.CompilerParams(needs_layout_passes=False),
        scratch_types=[
            pltpu.VMEM((2, 2, CH), jnp.int32),
            pltpu.VMEM((CH, D), f32),
            pltpu.VMEM((CH, D), f32),
            pltpu.VMEM((CH, D), f32),
            pltpu.VMEM((CH, D), f32),
            pltpu.VMEM((CH,), f32),
            pltpu.VMEM((CH,), f32),
            pltpu.VMEM((8, 16), f32),
            pltpu.VMEM((16,), f32),
            pltpu.VMEM((APW,), jnp.int32),
            pltpu.VMEM((APW, D), f32),
            pltpu.VMEM((16, 16), f32),
            pltpu.SemaphoreType.DMA,
            pltpu.SemaphoreType.DMA,
            pltpu.SemaphoreType.DMA,
            pltpu.SemaphoreType.DMA,
            pltpu.SemaphoreType.DMA,
            pltpu.SemaphoreType.DMA,
            pltpu.SemaphoreType.DMA,
        ],
    )


def _sc_attack_body(p1_hbm, p2_hbm, asrc_hbm, adst_hbm, wa2_hbm, ba2_hbm,
                    x_hbm, aidx_hbm, q_hbm, arows_hbm,
                    idx, p1r0, p1r1, p2r0, p2r1, qout0, qout1,
                    wa2v, ba2v, aidx, arows, tbuf,
                    s1a, s1b, s2a, s2b, sqa, sqb, sal):
    cid = lax.axis_index("c")
    sid = lax.axis_index("s")
    w = cid * NS + sid
    ebase = w * EPW
    p1r = (p1r0, p1r1)
    p2r = (p2r0, p2r1)
    qout = (qout0, qout1)
    sem1 = (s1a, s1b)
    sem2 = (s2a, s2b)
    semq = (sqa, sqb)

    def issue(c, b):
        base = ebase + c * CH
        pltpu.sync_copy(asrc_hbm.at[pl.ds(base, CH)], idx.at[b, 0])
        pltpu.sync_copy(adst_hbm.at[pl.ds(base, CH)], idx.at[b, 1])
        pltpu.async_copy(p1_hbm.at[idx.at[b, 0]], p1r[b], sem1[b])
        pltpu.async_copy(p2_hbm.at[idx.at[b, 1]], p2r[b], sem2[b])

    def wait(b):
        pltpu.make_async_copy(p1_hbm.at[idx.at[b, 0]], p1r[b], sem1[b]).wait()
        pltpu.make_async_copy(p2_hbm.at[idx.at[b, 1]], p2r[b], sem2[b]).wait()

    issue(0, 0)

    pltpu.sync_copy(wa2_hbm, wa2v)
    pltpu.sync_copy(ba2_hbm, ba2v)

    pltpu.sync_copy(aidx_hbm.at[pl.ds(w * APW, APW)], aidx)
    pltpu.async_copy(x_hbm.at[aidx], arows, sal).wait()
    pltpu.sync_copy(arows, arows_hbm.at[pl.ds(w * APW, APW)])

    lane = lax.iota(jnp.int32, 16)

    def compute(c, b):
        @pl.when(c >= 2)
        def _():
            pltpu.make_async_copy(qout[b], q_hbm.at[pl.ds(ebase, CH)],
                                  semq[b]).wait()

        def group(gi, _):
            e0 = gi * 16

            def edge(k, _):
                e = e0 + k
                acc = ba2v[...]
                for jj in range(8):
                    a = p1r[b][e, pl.ds(16 * jj, 16)]
                    bb = p2r[b][e, pl.ds(16 * jj, 16)]
                    t = jnp.maximum(a + bb, 0.0)
                    u = plsc.bitcast(t, jnp.uint32)
                    rnd = (u + jnp.uint32(0x8000)) & jnp.uint32(0xFFFF0000)
                    t = plsc.bitcast(rnd, f32)
                    acc = acc + t * wa2v[jj]
                plsc.store_scatter(tbuf, [lane, lane * 0 + k], acc)
                return 0

            lax.fori_loop(0, 16, edge, 0)
            acc16 = tbuf[0]
            for r in range(1, 16):
                acc16 = acc16 + tbuf[r]
            qout[b][pl.ds(e0, 16)] = acc16
            return 0

        lax.fori_loop(0, CH // 16, group, 0)
        pltpu.async_copy(qout[b], q_hbm.at[pl.ds(ebase + c * CH, CH)],
                         semq[b])

    def step(i, _):
        issue(2 * i + 1, 1)
        wait(0)
        compute(2 * i, 0)

        @pl.when(i < NIT - 1)
        def _():
            issue(2 * i + 2, 0)

        wait(1)
        compute(2 * i + 1, 1)
        return 0

    lax.fori_loop(0, NIT, step, 0)
    pltpu.make_async_copy(qout0, q_hbm.at[pl.ds(ebase, CH)], sqa).wait()
    pltpu.make_async_copy(qout1, q_hbm.at[pl.ds(ebase, CH)], sqb).wait()


BR = 2000
NBLK = N // BR


def _k1_body(x_ref, w_ref, y_ref):
    y_ref[...] = jnp.maximum(_mm(x_ref[...], w_ref[...]), 0.0)


def _tc_relu_matmul(x, w):
    return pl.pallas_call(
        _k1_body,
        grid=(NBLK,),
        in_specs=[pl.BlockSpec((BR, D), lambda i: (i, 0)),
                  pl.BlockSpec((D, D), lambda i: (0, 0))],
        out_specs=pl.BlockSpec((BR, D), lambda i: (i, 0)),
        out_shape=jax.ShapeDtypeStruct((N, D), f32),
    )(x, w)


def _ln_relu(h, gam, bet):
    mu = jnp.mean(h, axis=-1, keepdims=True)
    var = jnp.mean((h - mu) ** 2, axis=-1, keepdims=True)
    xn = (h - mu) / jnp.sqrt(var + 1e-5) * gam + bet
    return jnp.maximum(xn, 0.0)


def _k2_body(x_ref, agg_ref, wn1, wn2, bnr, gam, bet, wm, x1_ref, y1_ref):
    agg = agg_ref[0] + agg_ref[1]
    h = _mm(x_ref[...], wn1[...]) + _mm(agg, wn2[...]) + bnr[...]
    x1 = _ln_relu(h, gam[...], bet[...])
    x1_ref[...] = x1
    y1_ref[...] = jnp.maximum(_mm(x1, wm[...]), 0.0)


def _tc_layer_fused(x, aggs, wn1, wn2, bnr, gam, bet, wm):
    return pl.pallas_call(
        _k2_body,
        grid=(NBLK,),
        in_specs=[pl.BlockSpec((BR, D), lambda i: (i, 0)),
                  pl.BlockSpec((NC, BR, D), lambda i: (0, i, 0)),
                  pl.BlockSpec((D, D), lambda i: (0, 0)),
                  pl.BlockSpec((D, D), lambda i: (0, 0)),
                  pl.BlockSpec((1, D), lambda i: (0, 0)),
                  pl.BlockSpec((1, D), lambda i: (0, 0)),
                  pl.BlockSpec((1, D), lambda i: (0, 0)),
                  pl.BlockSpec((D, D), lambda i: (0, 0))],
        out_specs=[pl.BlockSpec((BR, D), lambda i: (i, 0)),
                   pl.BlockSpec((BR, D), lambda i: (i, 0))],
        out_shape=[jax.ShapeDtypeStruct((N, D), f32),
                   jax.ShapeDtypeStruct((N, D), f32)],
    )(x, aggs, wn1, wn2, bnr, gam, bet, wm)


def _k3_body(x_ref, agg_ref, wn1, wn2, bnr, gam, bet, x2_ref, ps_ref):
    agg = agg_ref[0] + agg_ref[1]
    h = _mm(x_ref[...], wn1[...]) + _mm(agg, wn2[...]) + bnr[...]
    x2 = _ln_relu(h, gam[...], bet[...])
    x2_ref[...] = x2
    ps_ref[...] = jnp.sum(x2, axis=0, keepdims=True)[None]


def _tc_layer_final(x, aggs, wn1, wn2, bnr, gam, bet):
    return pl.pallas_call(
        _k3_body,
        grid=(NBLK,),
        in_specs=[pl.BlockSpec((BR, D), lambda i: (i, 0)),
                  pl.BlockSpec((NC, BR, D), lambda i: (0, i, 0)),
                  pl.BlockSpec((D, D), lambda i: (0, 0)),
                  pl.BlockSpec((D, D), lambda i: (0, 0)),
                  pl.BlockSpec((1, D), lambda i: (0, 0)),
                  pl.BlockSpec((1, D), lambda i: (0, 0)),
                  pl.BlockSpec((1, D), lambda i: (0, 0))],
        out_specs=[pl.BlockSpec((BR, D), lambda i: (i, 0)),
                   pl.BlockSpec((1, 1, D), lambda i: (i, 0, 0))],
        out_shape=[jax.ShapeDtypeStruct((N, D), f32),
                   jax.ShapeDtypeStruct((NBLK, 1, D), f32)],
    )(x, aggs, wn1, wn2, bnr, gam, bet)


def _k4_body(x_ref, ws, wd, c_ref, p1_ref, p2_ref):
    p1_ref[...] = _mm(x_ref[...], ws[...]) + c_ref[...]
    p2_ref[...] = _mm(x_ref[...], wd[...])


def _tc_p(x, ws, wd, c):
    return pl.pallas_call(
        _k4_body,
        grid=(NBLK,),
        in_specs=[pl.BlockSpec((BR, D), lambda i: (i, 0)),
                  pl.BlockSpec((D, D), lambda i: (0, 0)),
                  pl.BlockSpec((D, D), lambda i: (0, 0)),
                  pl.BlockSpec((1, D), lambda i: (0, 0))],
        out_specs=[pl.BlockSpec((BR, D), lambda i: (i, 0)),
                   pl.BlockSpec((BR, D), lambda i: (i, 0))],
        out_shape=[jax.ShapeDtypeStruct((N, D), f32),
                   jax.ShapeDtypeStruct((N, D), f32)],
    )(x, ws, wd, c)


def _k5_body(ax_ref, wmv1, cmv, wh1, chh, w2m, w2h, brow, out_ref):
    ax = ax_ref[...]
    hm = jnp.maximum(_mm(ax, wmv1[...]) + cmv[...], 0.0)
    hh = jnp.maximum(_mm(ax, wh1[...]) + chh[...], 0.0)
    out_ref[...] = _mm(hm, w2m[...]) + _mm(hh, w2h[...]) + brow[...]


def _tc_ally(arows, wmv1, cmv, wh1, chh, w2m, w2h, brow):
    return pl.pallas_call(
        _k5_body,
        grid=(1,),
        in_specs=[pl.BlockSpec((APAD, D), lambda i: (0, 0)),
                  pl.BlockSpec((D, D), lambda i: (0, 0)),
                  pl.BlockSpec((1, D), lambda i: (0, 0)),
                  pl.BlockSpec((D, D), lambda i: (0, 0)),
                  pl.BlockSpec((1, D), lambda i: (0, 0)),
                  pl.BlockSpec((D, D), lambda i: (0, 0)),
                  pl.BlockSpec((D, D), lambda i: (0, 0)),
                  pl.BlockSpec((1, D), lambda i: (0, 0))],
        out_specs=pl.BlockSpec((APAD, D), lambda i: (0, 0)),
        out_shape=jax.ShapeDtypeStruct((APAD, D), f32),
    )(arows, wmv1, cmv, wh1, chh, w2m, w2h, brow)


def kernel(node_feature, global_feature, Wm, Wn, bn, gamma, beta, Wg, bg,
           Wmv1, bmv1, Wmv2, bmv2, Wh1, bh1, Wh2, bh2, Wa1, ba1, Wa2, ba2,
           edge_index, attack_edge_index, ally_indices):
    adst = attack_edge_index[1]

    npad = EPAD - E
    spread = jnp.arange(npad, dtype=jnp.int32) * 13 % N
    src_pad = jnp.concatenate([edge_index[0], spread])
    dst_pad = jnp.concatenate(
        [edge_index[1], N + (jnp.arange(npad, dtype=jnp.int32) % (NP - N))])
    asrc_pad = jnp.concatenate([attack_edge_index[0], spread])
    adst_pad = jnp.concatenate([attack_edge_index[1], spread])

    y0 = _tc_relu_matmul(node_feature, Wm[0])
    aggs0 = _sc_segment_sum_kernel()(y0, src_pad, dst_pad)
    x1, y1 = _tc_layer_fused(node_feature, aggs0,
                             Wn[0][:D], Wn[0][D:], bn[0].reshape(1, D),
                             gamma[0].reshape(1, D), beta[0].reshape(1, D),
                             Wm[1])
    aggs1 = _sc_segment_sum_kernel()(y1, src_pad, dst_pad)
    x2, psum = _tc_layer_final(x1, aggs1,
                               Wn[1][:D], Wn[1][D:], bn[1].reshape(1, D),
                               gamma[1].reshape(1, D), beta[1].reshape(1, D))

    pooled = jnp.sum(psum[:, 0, :], axis=0, keepdims=True) / 10000.0
    g = jax.nn.relu(
        _mm(jnp.concatenate([global_feature, pooled], axis=-1), Wg) + bg)

    c = _mm(g, Wa1[D:D + G]) + _mm(g, Wa1[2 * D + G:]) + ba1.reshape(1, D)
    P1, P2 = _tc_p(x2, Wa1[:D], Wa1[D + G:2 * D + G], c)

    aidx_pad = jnp.concatenate(
        [ally_indices, jnp.zeros((APAD - A,), jnp.int32)])
    wa2_r = Wa2.astype(jnp.bfloat16).astype(f32).reshape(8, 16)
    ba2v = jnp.full((16,), ba2[0] / 16.0, f32)
    q_full, arows = _sc_attack_kernel()(P1, P2, asrc_pad, adst_pad,
                                        wa2_r, ba2v, x2, aidx_pad)
    q_attack = q_full[:E]

    cmv = _mm(g, Wmv1[D:]) + bmv1.reshape(1, H)
    chh = _mm(g, Wh1[D:]) + bh1.reshape(1, H)
    w2m = jnp.zeros((H, D), f32).at[:, :4].set(Wmv2)
    w2h = jnp.zeros((H, D), f32).at[:, 4:5].set(Wh2)
    brow = jnp.zeros((1, D), f32).at[0, :4].set(bmv2).at[0, 4].set(bh2[0])
    out5 = _tc_ally(arows, Wmv1[:D], cmv, Wh1[:D], chh, w2m, w2h, brow)

    q_move = out5[:A, :4]
    q_hold = out5[:A, 4]
    return q_move, q_hold, q_attack, adst
